# baseline (speedup 1.0000x reference)
import jax
import jax.numpy as jnp
from jax.experimental import pallas as pl
from jax.experimental.pallas import tpu as pltpu

N = 100000
D = 128
K = 16
OUT = 256
BN1 = 5000
BN2 = 5000
NB1 = N // BN1
NB2 = N // BN2


def _reduce_kernel(ev_ref, x_ref, xf_ref, acc_ref):
    i = pl.program_id(0)

    @pl.when(i == 0)
    def _init():
        acc_ref[...] = jnp.zeros_like(acc_ref)

    acc_ref[...] += jax.lax.dot_general(
        ev_ref[...], x_ref[...], (((0,), (0,)), ((), ())),
        preferred_element_type=jnp.float32)

    @pl.when(i == NB1 - 1)
    def _finalize():
        xf_ref[...] = acc_ref[...]


def _mlp_kernel(xf_ref, evals_ref, w1_ref, b1_ref, w2_ref, b2_ref,
                mask_ref, wp_ref, wc_ref):
    h = jax.lax.dot_general(
        w1_ref[...], evals_ref[...], (((0,), (0,)), ((), ())),
        preferred_element_type=jnp.float32)
    h = jnp.maximum(h + b1_ref[...], 0.0)
    f = jax.lax.dot_general(
        w2_ref[...], h, (((0,), (0,)), ((), ())),
        preferred_element_type=jnp.float32)
    f = jnp.tanh(f + b2_ref[...]) * mask_ref[...]
    x_filt = f * xf_ref[...]
    wc_ref[...] = jnp.dot(x_filt, wp_ref[...],
                          preferred_element_type=jnp.float32)


def _stream_kernel(ev_ref, wc_ref, bp_ref, out_ref):
    out_ref[...] = jnp.dot(ev_ref[...], wc_ref[...],
                           preferred_element_type=jnp.float32) + bp_ref[...]


def kernel(x, eigvecs, eigvals, eig_mask, W1, b1, W2, b2, Wp, bp):
    evals_c = eigvals.reshape(K, 1)
    b1_c = b1.reshape(K // 2, 1)
    b2_c = b2.reshape(K, 1)
    mask_c = eig_mask.astype(jnp.float32).reshape(K, 1)
    bp_r = bp.reshape(1, OUT)

    x_freq = pl.pallas_call(
        _reduce_kernel,
        grid=(NB1,),
        in_specs=[
            pl.BlockSpec((BN1, K), lambda i: (i, 0)),
            pl.BlockSpec((BN1, D), lambda i: (i, 0)),
        ],
        out_specs=pl.BlockSpec((K, D), lambda i: (0, 0)),
        out_shape=jax.ShapeDtypeStruct((K, D), jnp.float32),
        scratch_shapes=[pltpu.VMEM((K, D), jnp.float32)],
    )(eigvecs, x)

    w_comb = pl.pallas_call(
        _mlp_kernel,
        out_shape=jax.ShapeDtypeStruct((K, OUT), jnp.float32),
    )(x_freq, evals_c, W1, b1_c, W2, b2_c, mask_c, Wp)

    out = pl.pallas_call(
        _stream_kernel,
        grid=(NB2,),
        in_specs=[
            pl.BlockSpec((BN2, K), lambda i: (i, 0)),
            pl.BlockSpec((K, OUT), lambda i: (0, 0)),
            pl.BlockSpec((1, OUT), lambda i: (0, 0)),
        ],
        out_specs=pl.BlockSpec((BN2, OUT), lambda i: (i, 0)),
        out_shape=jax.ShapeDtypeStruct((N, OUT), jnp.float32),
    )(eigvecs, w_comb, bp_r)

    return out


# speedup vs baseline: 1.0028x; 1.0028x over previous
import jax
import jax.numpy as jnp
from jax.experimental import pallas as pl
from jax.experimental.pallas import tpu as pltpu

N = 100000
D = 128
K = 16
OUT = 256

BN1 = 4000
NB1 = N // BN1
L1 = 4
R1 = L1 + 1

BN2 = 4000
NB2 = N // BN2
R2 = 6


def _reduce_kernel(ev_ref, x_hbm, xf_ref, acc_ref, buf, sems):
    i = pl.program_id(0)

    @pl.when(i == 0)
    def _prologue():
        acc_ref[...] = jnp.zeros_like(acc_ref)
        for j in range(L1):
            pltpu.make_async_copy(
                x_hbm.at[pl.ds(j * BN1, BN1), :], buf.at[j], sems.at[j]
            ).start()

    @pl.when(i + L1 < NB1)
    def _issue_ahead():
        slot = jax.lax.rem(i + L1, R1)
        pltpu.make_async_copy(
            x_hbm.at[pl.ds((i + L1) * BN1, BN1), :], buf.at[slot],
            sems.at[slot],
        ).start()

    slot_i = jax.lax.rem(i, R1)
    pltpu.make_async_copy(
        x_hbm.at[pl.ds(i * BN1, BN1), :], buf.at[slot_i], sems.at[slot_i]
    ).wait()

    acc_ref[...] += jax.lax.dot_general(
        ev_ref[...], buf[slot_i], (((0,), (0,)), ((), ())),
        preferred_element_type=jnp.float32)

    @pl.when(i == NB1 - 1)
    def _finalize():
        xf_ref[...] = acc_ref[...]


def _mlp_kernel(xf_ref, evals_ref, w1_ref, b1_ref, w2_ref, b2_ref,
                mask_ref, wp_ref, wc_ref):
    h = jax.lax.dot_general(
        w1_ref[...], evals_ref[...], (((0,), (0,)), ((), ())),
        preferred_element_type=jnp.float32)
    h = jnp.maximum(h + b1_ref[...], 0.0)
    f = jax.lax.dot_general(
        w2_ref[...], h, (((0,), (0,)), ((), ())),
        preferred_element_type=jnp.float32)
    f = jnp.tanh(f + b2_ref[...]) * mask_ref[...]
    x_filt = f * xf_ref[...]
    wc_ref[...] = jnp.dot(x_filt, wp_ref[...],
                          preferred_element_type=jnp.float32)


def _stream_kernel(ev_ref, wc_ref, bp_ref, out_hbm, buf, sems):
    i = pl.program_id(0)
    slot = jax.lax.rem(i, R2)

    @pl.when(i >= R2)
    def _reclaim():
        pltpu.make_async_copy(
            buf.at[slot], out_hbm.at[pl.ds((i - R2) * BN2, BN2), :],
            sems.at[slot],
        ).wait()

    buf[slot] = jnp.dot(ev_ref[...], wc_ref[...],
                        preferred_element_type=jnp.float32) + bp_ref[...]
    pltpu.make_async_copy(
        buf.at[slot], out_hbm.at[pl.ds(i * BN2, BN2), :], sems.at[slot]
    ).start()

    @pl.when(i == NB2 - 1)
    def _drain():
        for j in range(R2):
            s = NB2 - R2 + j
            pltpu.make_async_copy(
                buf.at[s % R2], out_hbm.at[pl.ds(s * BN2, BN2), :],
                sems.at[s % R2],
            ).wait()


def kernel(x, eigvecs, eigvals, eig_mask, W1, b1, W2, b2, Wp, bp):
    evals_c = eigvals.reshape(K, 1)
    b1_c = b1.reshape(K // 2, 1)
    b2_c = b2.reshape(K, 1)
    mask_c = eig_mask.astype(jnp.float32).reshape(K, 1)
    bp_r = bp.reshape(1, OUT)

    x_freq = pl.pallas_call(
        _reduce_kernel,
        grid=(NB1,),
        in_specs=[
            pl.BlockSpec((BN1, K), lambda i: (i, 0)),
            pl.BlockSpec(memory_space=pltpu.MemorySpace.HBM),
        ],
        out_specs=pl.BlockSpec((K, D), lambda i: (0, 0)),
        out_shape=jax.ShapeDtypeStruct((K, D), jnp.float32),
        scratch_shapes=[
            pltpu.VMEM((K, D), jnp.float32),
            pltpu.VMEM((R1, BN1, D), jnp.float32),
            pltpu.SemaphoreType.DMA((R1,)),
        ],
        compiler_params=pltpu.CompilerParams(
            dimension_semantics=("arbitrary",)),
    )(eigvecs, x)

    w_comb = pl.pallas_call(
        _mlp_kernel,
        out_shape=jax.ShapeDtypeStruct((K, OUT), jnp.float32),
    )(x_freq, evals_c, W1, b1_c, W2, b2_c, mask_c, Wp)

    out = pl.pallas_call(
        _stream_kernel,
        grid=(NB2,),
        in_specs=[
            pl.BlockSpec((BN2, K), lambda i: (i, 0)),
            pl.BlockSpec((K, OUT), lambda i: (0, 0)),
            pl.BlockSpec((1, OUT), lambda i: (0, 0)),
        ],
        out_specs=pl.BlockSpec(memory_space=pltpu.MemorySpace.HBM),
        out_shape=jax.ShapeDtypeStruct((N, OUT), jnp.float32),
        scratch_shapes=[
            pltpu.VMEM((R2, BN2, OUT), jnp.float32),
            pltpu.SemaphoreType.DMA((R2,)),
        ],
        compiler_params=pltpu.CompilerParams(
            dimension_semantics=("arbitrary",)),
    )(eigvecs, w_comb, bp_r)

    return out


# speedup vs baseline: 1.1972x; 1.1939x over previous
import jax
import jax.numpy as jnp
from jax.experimental import pallas as pl
from jax.experimental.pallas import tpu as pltpu

N = 100000
D = 128
K = 16
OUT = 256

BN1 = 4000
NB1 = N // BN1
L1 = 4
R1 = L1 + 1

BN2 = 4000
NB2 = N // BN2
R2 = 6


def _reduce_kernel(ev_ref, x_hbm, xf_ref, acc_ref, buf, sems):
    i = pl.program_id(0)

    @pl.when(i == 0)
    def _prologue():
        acc_ref[...] = jnp.zeros_like(acc_ref)
        for j in range(L1):
            pltpu.make_async_copy(
                x_hbm.at[pl.ds(j * BN1, BN1), :], buf.at[j], sems.at[j]
            ).start()

    @pl.when(i + L1 < NB1)
    def _issue_ahead():
        slot = jax.lax.rem(i + L1, R1)
        pltpu.make_async_copy(
            x_hbm.at[pl.ds((i + L1) * BN1, BN1), :], buf.at[slot],
            sems.at[slot],
        ).start()

    slot_i = jax.lax.rem(i, R1)
    pltpu.make_async_copy(
        x_hbm.at[pl.ds(i * BN1, BN1), :], buf.at[slot_i], sems.at[slot_i]
    ).wait()

    acc_ref[...] += jax.lax.dot_general(
        ev_ref[...], buf[slot_i].astype(jnp.bfloat16),
        (((0,), (0,)), ((), ())),
        preferred_element_type=jnp.float32)

    @pl.when(i == NB1 - 1)
    def _finalize():
        xf_ref[...] = acc_ref[...]


def _mlp_kernel(xf_ref, evals_ref, w1_ref, b1_ref, w2_ref, b2_ref,
                mask_ref, wp_ref, wc_ref):
    h = jax.lax.dot_general(
        w1_ref[...], evals_ref[...], (((0,), (0,)), ((), ())),
        preferred_element_type=jnp.float32)
    h = jnp.maximum(h + b1_ref[...], 0.0)
    f = jax.lax.dot_general(
        w2_ref[...], h, (((0,), (0,)), ((), ())),
        preferred_element_type=jnp.float32)
    f = jnp.tanh(f + b2_ref[...]) * mask_ref[...]
    x_filt = f * xf_ref[...]
    wc_ref[...] = jnp.dot(x_filt, wp_ref[...],
                          preferred_element_type=jnp.float32
                          ).astype(jnp.bfloat16)


def _stream_kernel(ev_ref, wc_ref, bp_ref, out_hbm, buf, sems):
    i = pl.program_id(0)
    slot = jax.lax.rem(i, R2)

    @pl.when(i >= R2)
    def _reclaim():
        pltpu.make_async_copy(
            buf.at[slot], out_hbm.at[pl.ds((i - R2) * BN2, BN2), :],
            sems.at[slot],
        ).wait()

    buf[slot] = jnp.dot(ev_ref[...], wc_ref[...],
                        preferred_element_type=jnp.float32) + bp_ref[...]
    pltpu.make_async_copy(
        buf.at[slot], out_hbm.at[pl.ds(i * BN2, BN2), :], sems.at[slot]
    ).start()

    @pl.when(i == NB2 - 1)
    def _drain():
        for j in range(R2):
            s = NB2 - R2 + j
            pltpu.make_async_copy(
                buf.at[s % R2], out_hbm.at[pl.ds(s * BN2, BN2), :],
                sems.at[s % R2],
            ).wait()


def kernel(x, eigvecs, eigvals, eig_mask, W1, b1, W2, b2, Wp, bp):
    ev_bf = eigvecs.astype(jnp.bfloat16)
    evals_c = eigvals.reshape(K, 1)
    b1_c = b1.reshape(K // 2, 1)
    b2_c = b2.reshape(K, 1)
    mask_c = eig_mask.astype(jnp.float32).reshape(K, 1)
    bp_r = bp.reshape(1, OUT)

    x_freq = pl.pallas_call(
        _reduce_kernel,
        grid=(NB1,),
        in_specs=[
            pl.BlockSpec((BN1, K), lambda i: (i, 0)),
            pl.BlockSpec(memory_space=pltpu.MemorySpace.HBM),
        ],
        out_specs=pl.BlockSpec((K, D), lambda i: (0, 0)),
        out_shape=jax.ShapeDtypeStruct((K, D), jnp.float32),
        scratch_shapes=[
            pltpu.VMEM((K, D), jnp.float32),
            pltpu.VMEM((R1, BN1, D), jnp.float32),
            pltpu.SemaphoreType.DMA((R1,)),
        ],
        compiler_params=pltpu.CompilerParams(
            dimension_semantics=("arbitrary",)),
    )(ev_bf, x)

    w_comb = pl.pallas_call(
        _mlp_kernel,
        out_shape=jax.ShapeDtypeStruct((K, OUT), jnp.bfloat16),
    )(x_freq, evals_c, W1, b1_c, W2, b2_c, mask_c, Wp)

    out = pl.pallas_call(
        _stream_kernel,
        grid=(NB2,),
        in_specs=[
            pl.BlockSpec((BN2, K), lambda i: (i, 0)),
            pl.BlockSpec((K, OUT), lambda i: (0, 0)),
            pl.BlockSpec((1, OUT), lambda i: (0, 0)),
        ],
        out_specs=pl.BlockSpec(memory_space=pltpu.MemorySpace.HBM),
        out_shape=jax.ShapeDtypeStruct((N, OUT), jnp.float32),
        scratch_shapes=[
            pltpu.VMEM((R2, BN2, OUT), jnp.float32),
            pltpu.SemaphoreType.DMA((R2,)),
        ],
        compiler_params=pltpu.CompilerParams(
            dimension_semantics=("arbitrary",)),
    )(ev_bf, w_comb, bp_r)

    return out


# speedup vs baseline: 1.5448x; 1.2904x over previous
import jax
import jax.numpy as jnp
from jax.experimental import pallas as pl
from jax.experimental.pallas import tpu as pltpu

N = 100000
D = 128
K = 16
OUT = 256

BN1 = 4000
NB1 = N // BN1
L1 = 4
R1 = L1 + 1

BN2 = 4000
NB2 = N // BN2
R2 = 6
assert BN1 == BN2


def _reduce_kernel(evt_ref, x_hbm, xf_ref, acc_ref, buf, sems):
    i = pl.program_id(0)

    @pl.when(i == 0)
    def _prologue():
        acc_ref[...] = jnp.zeros_like(acc_ref)
        for j in range(L1):
            pltpu.make_async_copy(
                x_hbm.at[pl.ds(j * BN1, BN1), :], buf.at[j], sems.at[j]
            ).start()

    @pl.when(i + L1 < NB1)
    def _issue_ahead():
        slot = jax.lax.rem(i + L1, R1)
        pltpu.make_async_copy(
            x_hbm.at[pl.ds((i + L1) * BN1, BN1), :], buf.at[slot],
            sems.at[slot],
        ).start()

    slot_i = jax.lax.rem(i, R1)
    pltpu.make_async_copy(
        x_hbm.at[pl.ds(i * BN1, BN1), :], buf.at[slot_i], sems.at[slot_i]
    ).wait()

    acc_ref[...] += jax.lax.dot_general(
        evt_ref[i], buf[slot_i].astype(jnp.bfloat16),
        (((1,), (0,)), ((), ())),
        preferred_element_type=jnp.float32)

    @pl.when(i == NB1 - 1)
    def _finalize():
        xf_ref[...] = acc_ref[...]


def _mlp_kernel(xf_ref, evals_ref, w1_ref, b1_ref, w2_ref, b2_ref,
                mask_ref, wp_ref, wc_ref):
    h = jax.lax.dot_general(
        w1_ref[...], evals_ref[...], (((0,), (0,)), ((), ())),
        preferred_element_type=jnp.float32)
    h = jnp.maximum(h + b1_ref[...], 0.0)
    f = jax.lax.dot_general(
        w2_ref[...], h, (((0,), (0,)), ((), ())),
        preferred_element_type=jnp.float32)
    f = jnp.tanh(f + b2_ref[...]) * mask_ref[...]
    x_filt = f * xf_ref[...]
    wc_ref[...] = jnp.dot(x_filt, wp_ref[...],
                          preferred_element_type=jnp.float32
                          ).astype(jnp.bfloat16)


def _stream_kernel(evt_ref, wc_ref, bp_ref, out_hbm, buf, sems):
    i = pl.program_id(0)
    slot = jax.lax.rem(i, R2)

    @pl.when(i >= R2)
    def _reclaim():
        pltpu.make_async_copy(
            buf.at[slot], out_hbm.at[pl.ds((i - R2) * BN2, BN2), :],
            sems.at[slot],
        ).wait()

    buf[slot] = jax.lax.dot_general(
        evt_ref[i], wc_ref[...],
        (((0,), (0,)), ((), ())),
        preferred_element_type=jnp.float32) + bp_ref[...]
    pltpu.make_async_copy(
        buf.at[slot], out_hbm.at[pl.ds(i * BN2, BN2), :], sems.at[slot]
    ).start()

    @pl.when(i == NB2 - 1)
    def _drain():
        for j in range(R2):
            s = NB2 - R2 + j
            pltpu.make_async_copy(
                buf.at[s % R2], out_hbm.at[pl.ds(s * BN2, BN2), :],
                sems.at[s % R2],
            ).wait()


def kernel(x, eigvecs, eigvals, eig_mask, W1, b1, W2, b2, Wp, bp):
    evt = (eigvecs.astype(jnp.bfloat16).T
           .reshape(K, NB1, BN1).transpose(1, 0, 2))
    evals_c = eigvals.reshape(K, 1)
    b1_c = b1.reshape(K // 2, 1)
    b2_c = b2.reshape(K, 1)
    mask_c = eig_mask.astype(jnp.float32).reshape(K, 1)
    bp_r = bp.reshape(1, OUT)

    x_freq = pl.pallas_call(
        _reduce_kernel,
        grid=(NB1,),
        in_specs=[
            pl.BlockSpec((NB1, K, BN1), lambda i: (0, 0, 0)),
            pl.BlockSpec(memory_space=pltpu.MemorySpace.HBM),
        ],
        out_specs=pl.BlockSpec((K, D), lambda i: (0, 0)),
        out_shape=jax.ShapeDtypeStruct((K, D), jnp.float32),
        scratch_shapes=[
            pltpu.VMEM((K, D), jnp.float32),
            pltpu.VMEM((R1, BN1, D), jnp.float32),
            pltpu.SemaphoreType.DMA((R1,)),
        ],
        compiler_params=pltpu.CompilerParams(
            dimension_semantics=("arbitrary",)),
    )(evt, x)

    w_comb = pl.pallas_call(
        _mlp_kernel,
        out_shape=jax.ShapeDtypeStruct((K, OUT), jnp.bfloat16),
    )(x_freq, evals_c, W1, b1_c, W2, b2_c, mask_c, Wp)

    out = pl.pallas_call(
        _stream_kernel,
        grid=(NB2,),
        in_specs=[
            pl.BlockSpec((NB2, K, BN2), lambda i: (0, 0, 0)),
            pl.BlockSpec((K, OUT), lambda i: (0, 0)),
            pl.BlockSpec((1, OUT), lambda i: (0, 0)),
        ],
        out_specs=pl.BlockSpec(memory_space=pltpu.MemorySpace.HBM),
        out_shape=jax.ShapeDtypeStruct((N, OUT), jnp.float32),
        scratch_shapes=[
            pltpu.VMEM((R2, BN2, OUT), jnp.float32),
            pltpu.SemaphoreType.DMA((R2,)),
        ],
        compiler_params=pltpu.CompilerParams(
            dimension_semantics=("arbitrary",)),
    )(evt, w_comb, bp_r)

    return out


# speedup vs baseline: 1.5867x; 1.0271x over previous
import jax
import jax.numpy as jnp
from jax.experimental import pallas as pl
from jax.experimental.pallas import tpu as pltpu

N = 100000
D = 128
K = 16
OUT = 256

BN1 = 4000
NB1 = N // BN1
L1 = 4
R1 = L1 + 1

BN2 = 4000
NB2 = N // BN2
R2 = 6
assert BN1 == BN2


def _reduce_kernel(evt_ref, x_hbm, evals_ref, w1_ref, b1_ref, w2_ref,
                   b2_ref, mask_ref, wp_ref, wc_ref, acc_ref, buf, sems):
    i = pl.program_id(0)

    @pl.when(i == 0)
    def _prologue():
        acc_ref[...] = jnp.zeros_like(acc_ref)
        for j in range(L1):
            pltpu.make_async_copy(
                x_hbm.at[pl.ds(j * BN1, BN1), :], buf.at[j], sems.at[j]
            ).start()

    @pl.when(i + L1 < NB1)
    def _issue_ahead():
        slot = jax.lax.rem(i + L1, R1)
        pltpu.make_async_copy(
            x_hbm.at[pl.ds((i + L1) * BN1, BN1), :], buf.at[slot],
            sems.at[slot],
        ).start()

    slot_i = jax.lax.rem(i, R1)
    pltpu.make_async_copy(
        x_hbm.at[pl.ds(i * BN1, BN1), :], buf.at[slot_i], sems.at[slot_i]
    ).wait()

    acc_ref[...] += jax.lax.dot_general(
        evt_ref[i], buf[slot_i].astype(jnp.bfloat16),
        (((1,), (0,)), ((), ())),
        preferred_element_type=jnp.float32)

    @pl.when(i == NB1 - 1)
    def _finalize():
        h = jax.lax.dot_general(
            w1_ref[...], evals_ref[...], (((0,), (0,)), ((), ())),
            preferred_element_type=jnp.float32)
        h = jnp.maximum(h + b1_ref[...], 0.0)
        f = jax.lax.dot_general(
            w2_ref[...], h, (((0,), (0,)), ((), ())),
            preferred_element_type=jnp.float32)
        f = jnp.tanh(f + b2_ref[...]) * mask_ref[...]
        x_filt = f * acc_ref[...]
        wc_ref[...] = jnp.dot(x_filt, wp_ref[...],
                              preferred_element_type=jnp.float32
                              ).astype(jnp.bfloat16)


def _stream_kernel(evt_ref, wc_ref, bp_ref, out_hbm, buf, sems):
    i = pl.program_id(0)
    slot = jax.lax.rem(i, R2)

    @pl.when(i >= R2)
    def _reclaim():
        pltpu.make_async_copy(
            buf.at[slot], out_hbm.at[pl.ds((i - R2) * BN2, BN2), :],
            sems.at[slot],
        ).wait()

    buf[slot] = jax.lax.dot_general(
        evt_ref[i], wc_ref[...],
        (((0,), (0,)), ((), ())),
        preferred_element_type=jnp.float32) + bp_ref[...]
    pltpu.make_async_copy(
        buf.at[slot], out_hbm.at[pl.ds(i * BN2, BN2), :], sems.at[slot]
    ).start()

    @pl.when(i == NB2 - 1)
    def _drain():
        for j in range(R2):
            s = NB2 - R2 + j
            pltpu.make_async_copy(
                buf.at[s % R2], out_hbm.at[pl.ds(s * BN2, BN2), :],
                sems.at[s % R2],
            ).wait()


def kernel(x, eigvecs, eigvals, eig_mask, W1, b1, W2, b2, Wp, bp):
    evt = (eigvecs.astype(jnp.bfloat16).T
           .reshape(K, NB1, BN1).transpose(1, 0, 2))
    evals_c = eigvals.reshape(K, 1)
    b1_c = b1.reshape(K // 2, 1)
    b2_c = b2.reshape(K, 1)
    mask_c = eig_mask.astype(jnp.float32).reshape(K, 1)
    bp_r = bp.reshape(1, OUT)

    w_comb = pl.pallas_call(
        _reduce_kernel,
        grid=(NB1,),
        in_specs=[
            pl.BlockSpec((NB1, K, BN1), lambda i: (0, 0, 0)),
            pl.BlockSpec(memory_space=pltpu.MemorySpace.HBM),
            pl.BlockSpec((K, 1), lambda i: (0, 0)),
            pl.BlockSpec((K, K // 2), lambda i: (0, 0)),
            pl.BlockSpec((K // 2, 1), lambda i: (0, 0)),
            pl.BlockSpec((K // 2, K), lambda i: (0, 0)),
            pl.BlockSpec((K, 1), lambda i: (0, 0)),
            pl.BlockSpec((K, 1), lambda i: (0, 0)),
            pl.BlockSpec((D, OUT), lambda i: (0, 0)),
        ],
        out_specs=pl.BlockSpec((K, OUT), lambda i: (0, 0)),
        out_shape=jax.ShapeDtypeStruct((K, OUT), jnp.bfloat16),
        scratch_shapes=[
            pltpu.VMEM((K, D), jnp.float32),
            pltpu.VMEM((R1, BN1, D), jnp.float32),
            pltpu.SemaphoreType.DMA((R1,)),
        ],
        compiler_params=pltpu.CompilerParams(
            dimension_semantics=("arbitrary",)),
    )(evt, x, evals_c, W1, b1_c, W2, b2_c, mask_c, Wp)

    out = pl.pallas_call(
        _stream_kernel,
        grid=(NB2,),
        in_specs=[
            pl.BlockSpec((NB2, K, BN2), lambda i: (0, 0, 0)),
            pl.BlockSpec((K, OUT), lambda i: (0, 0)),
            pl.BlockSpec((1, OUT), lambda i: (0, 0)),
        ],
        out_specs=pl.BlockSpec(memory_space=pltpu.MemorySpace.HBM),
        out_shape=jax.ShapeDtypeStruct((N, OUT), jnp.float32),
        scratch_shapes=[
            pltpu.VMEM((R2, BN2, OUT), jnp.float32),
            pltpu.SemaphoreType.DMA((R2,)),
        ],
        compiler_params=pltpu.CompilerParams(
            dimension_semantics=("arbitrary",)),
    )(evt, w_comb, bp_r)

    return out
